# baseline (speedup 1.0000x reference)
import functools

import jax
import jax.numpy as jnp
from jax import lax
from jax.experimental import pallas as pl
from jax.experimental.pallas import tpu as pltpu
from jax.experimental.pallas import tpu_sc as plsc

N = 10000
E = 320000
F = 128
FH = F // 2
NCLS = 64

NCORES = 2
NSUB = 16
EPS = E // NSUB
C = 128
NCHUNK = 160
NHALF = NCHUNK // 2
Z = 200
NZ = N // Z
CW = 16

_mesh = plsc.VectorSubcoreMesh(core_axis_name="c", subcore_axis_name="s")


def _zero_vmem(ref, nrow, ncol):
    def body(i, carry):
        ref[i // (ncol // 16), pl.ds((i % (ncol // 16)) * 16, 16)] = (
            jnp.zeros((16,), jnp.float32))
        return carry
    lax.fori_loop(0, nrow * (ncol // 16), body, None)


NBUF = 4


def _spmm_body(h_hbm, src_hbm, dst_hbm, out_hbm, idx_s, idx_d, bufs, zbuf,
               acc, sem_g, sem_s):
    c = lax.axis_index("c")
    s = lax.axis_index("s")

    _zero_vmem(zbuf, Z, FH)
    for t in range((NZ + NSUB - 1) // NSUB):
        j = s + NSUB * t

        @pl.when(j < NZ)
        def _():
            pltpu.sync_copy(zbuf, acc.at[pl.ds(j * Z, Z)])
    plsc.subcore_barrier()

    for half in range(2):
        pltpu.sync_copy(src_hbm.at[c, s, pl.ds(half * NHALF, NHALF)], idx_s)
        pltpu.sync_copy(dst_hbm.at[s, pl.ds(half * NHALF, NHALF)], idx_d)
        for b in range(NBUF):
            pltpu.async_copy(h_hbm.at[idx_s.at[b]], bufs[b], sem_g)

        def chunk(i, carry):
            for b in range(NBUF):
                k = NBUF * i + b
                pltpu.make_async_copy(h_hbm.at[idx_s.at[k]], bufs[b],
                                      sem_g).wait()
                pltpu.async_copy(bufs[b], acc.at[idx_d.at[k]], sem_s,
                                 add=True)
            for b in range(NBUF):
                k = NBUF * i + b
                pltpu.make_async_copy(bufs[b], acc.at[idx_d.at[k]],
                                      sem_s).wait()

                @pl.when(k + NBUF < NHALF)
                def _():
                    pltpu.async_copy(h_hbm.at[idx_s.at[k + NBUF]], bufs[b],
                                     sem_g)
            return carry
        lax.fori_loop(0, NHALF // NBUF, chunk, None)

    plsc.subcore_barrier()
    for t in range((NZ + NSUB - 1) // NSUB):
        j = s + NSUB * t

        @pl.when(j < NZ)
        def _():
            pltpu.sync_copy(acc.at[pl.ds(j * Z, Z)],
                            out_hbm.at[c, pl.ds(j * Z, Z)])


_sc_spmm = pl.kernel(
    _spmm_body,
    out_type=jax.ShapeDtypeStruct((NCORES, N, FH), jnp.float32),
    mesh=_mesh,
    scratch_types=[
        pltpu.VMEM((NHALF, C), jnp.int32),
        pltpu.VMEM((NHALF, C), jnp.int32),
        tuple(pltpu.VMEM((C, FH), jnp.float32) for _ in range(NBUF)),
        pltpu.VMEM((Z, FH), jnp.float32),
        pltpu.VMEM_SHARED((N + 8, FH), jnp.float32),
        pltpu.SemaphoreType.DMA,
        pltpu.SemaphoreType.DMA,
    ],
    compiler_params=pltpu.CompilerParams(use_tc_tiling_on_sc=False))

NCHUNK_D = 80
WIN = 8


def _cnt_body(dst_hbm, cnt_hbm, idx_d, ones, zbuf16, cnt_sh, sem):
    c = lax.axis_index("c")
    s = lax.axis_index("s")
    wid = c * NSUB + s
    pltpu.sync_copy(dst_hbm.at[wid], idx_d)

    _zero_vmem(zbuf16, Z, CW)

    def fill_ones(i, carry):
        ones[i, pl.ds(0, CW)] = jnp.ones((CW,), jnp.float32)
        return carry
    lax.fori_loop(0, C, fill_ones, None)
    for t in range((NZ + NSUB - 1) // NSUB):
        j = s + NSUB * t

        @pl.when(j < NZ)
        def _():
            pltpu.sync_copy(zbuf16, cnt_sh.at[pl.ds(j * Z, Z)])
    plsc.subcore_barrier()

    def chunk(k, carry):
        pltpu.async_copy(ones, cnt_sh.at[idx_d.at[k]], sem, add=True)

        @pl.when(k >= WIN)
        def _():
            pltpu.make_async_copy(ones, cnt_sh.at[idx_d.at[0]], sem).wait()
        return carry
    lax.fori_loop(0, NCHUNK_D, chunk, None)
    for _ in range(WIN):
        pltpu.make_async_copy(ones, cnt_sh.at[idx_d.at[0]], sem).wait()

    plsc.subcore_barrier()
    for t in range((NZ + NSUB - 1) // NSUB):
        j = s + NSUB * t

        @pl.when(j < NZ)
        def _():
            pltpu.sync_copy(cnt_sh.at[pl.ds(j * Z, Z)],
                            cnt_hbm.at[c, pl.ds(j * Z, Z)])


_sc_cnt = pl.kernel(
    _cnt_body,
    out_type=jax.ShapeDtypeStruct((NCORES, N, CW), jnp.float32),
    mesh=_mesh,
    scratch_types=[
        pltpu.VMEM((NCHUNK_D, C), jnp.int32),
        pltpu.VMEM((C, CW), jnp.float32),
        pltpu.VMEM((Z, CW), jnp.float32),
        pltpu.VMEM_SHARED((N + 8, CW), jnp.float32),
        pltpu.SemaphoreType.DMA,
    ],
    compiler_params=pltpu.CompilerParams(use_tc_tiling_on_sc=False))

R = 2000


def _dots(mean, h, wl_ref, wr_ref, bl_ref):
    return (jnp.dot(mean, wl_ref[...], preferred_element_type=jnp.float32,
                    precision=lax.Precision.DEFAULT)
            + jnp.dot(h, wr_ref[...], preferred_element_type=jnp.float32,
                      precision=lax.Precision.DEFAULT)
            + bl_ref[...])


def _combine_body(p_ref, c_ref, h_ref, wl_ref, wr_ref, bl_ref, o_ref):
    cnt = jnp.maximum(c_ref[0, :, 0:1] + c_ref[1, :, 0:1], 1.0)
    mean = jnp.concatenate([p_ref[0], p_ref[1]], axis=1) / cnt
    o_ref[...] = jnp.maximum(_dots(mean, h_ref[...], wl_ref, wr_ref, bl_ref),
                             0.0)


def _final_body(p_ref, c_ref, h_ref, wl_ref, wr_ref, bl_ref, wc_ref, bc_ref,
                o_ref):
    cnt = jnp.maximum(c_ref[0, :, 0:1] + c_ref[1, :, 0:1], 1.0)
    mean = jnp.concatenate([p_ref[0], p_ref[1]], axis=1) / cnt
    y = _dots(mean, h_ref[...], wl_ref, wr_ref, bl_ref)
    o_ref[...] = (jnp.dot(y, wc_ref[...], preferred_element_type=jnp.float32,
                          precision=lax.Precision.DEFAULT)
                  + bc_ref[...])


_common_specs = [
    pl.BlockSpec((NCORES, R, FH), lambda i: (0, i, 0)),
    pl.BlockSpec((NCORES, R, CW), lambda i: (0, i, 0)),
    pl.BlockSpec((R, F), lambda i: (i, 0)),
    pl.BlockSpec((F, F), lambda i: (0, 0)),
    pl.BlockSpec((F, F), lambda i: (0, 0)),
    pl.BlockSpec((1, F), lambda i: (0, 0)),
]

_combine_relu = pl.pallas_call(
    _combine_body,
    grid=(N // R,),
    in_specs=_common_specs,
    out_specs=pl.BlockSpec((R, F), lambda i: (i, 0)),
    out_shape=jax.ShapeDtypeStruct((N, F), jnp.float32),
)

_combine_final = pl.pallas_call(
    _final_body,
    grid=(N // R,),
    in_specs=_common_specs + [
        pl.BlockSpec((F, NCLS), lambda i: (0, 0)),
        pl.BlockSpec((1, NCLS), lambda i: (0, 0)),
    ],
    out_specs=pl.BlockSpec((R, NCLS), lambda i: (i, 0)),
    out_shape=jax.ShapeDtypeStruct((N, NCLS), jnp.float32),
)


def kernel(x, edge_index, Wl0, bl0, Wr0, Wl1, bl1, Wr1, Wl2, bl2, Wr2, Wc,
           bc):
    srcw = edge_index[0].reshape(NSUB, EPS)
    dstw = edge_index[1].reshape(NSUB, EPS)
    pad = NCHUNK * C - EPS
    srcp = jnp.pad(srcw, ((0, 0), (0, pad)))
    dstp = jnp.pad(dstw, ((0, 0), (0, pad)), constant_values=N)
    src2 = jnp.stack([2 * srcp, 2 * srcp + 1]).reshape(NCORES, NSUB,
                                                       NCHUNK, C)
    dst3 = dstp.reshape(NSUB, NCHUNK, C)
    dw = edge_index[1].reshape(NCORES * NSUB, E // (NCORES * NSUB))
    dwp = jnp.pad(dw, ((0, 0), (0, NCHUNK_D * C - dw.shape[1])),
                  constant_values=N).reshape(NCORES * NSUB, NCHUNK_D, C)

    cnt = _sc_cnt(dwp)
    p0 = _sc_spmm(x.reshape(2 * N, FH), src2, dst3)
    h1 = _combine_relu(p0, cnt, x, Wl0, Wr0, bl0.reshape(1, F))
    p1 = _sc_spmm(h1.reshape(2 * N, FH), src2, dst3)
    h2 = _combine_relu(p1, cnt, h1, Wl1, Wr1, bl1.reshape(1, F))
    p2 = _sc_spmm(h2.reshape(2 * N, FH), src2, dst3)
    return _combine_final(p2, cnt, h2, Wl2, Wr2, bl2.reshape(1, F), Wc,
                          bc.reshape(1, NCLS))


# speedup vs baseline: 2.9250x; 2.9250x over previous
import functools

import jax
import jax.numpy as jnp
from jax import lax
from jax.experimental import pallas as pl
from jax.experimental.pallas import tpu as pltpu
from jax.experimental.pallas import tpu_sc as plsc

N = 10000
E = 320000
F = 128
FH = F // 2
NCLS = 64

NCORES = 2
NSUB = 16
EPS = E // NSUB
C = 128
CS = 125
NCHUNK = EPS // CS
NHALF = NCHUNK // 2
Z = 200
NZ = N // Z
CW = 16

_mesh = plsc.VectorSubcoreMesh(core_axis_name="c", subcore_axis_name="s")


def _zero_vmem(ref, nrow, ncol):
    def body(i, carry):
        ref[i // (ncol // 16), pl.ds((i % (ncol // 16)) * 16, 16)] = (
            jnp.zeros((16,), jnp.float32))
        return carry
    lax.fori_loop(0, nrow * (ncol // 16), body, None)


NBUF = 4


def _spmm_body(h_hbm, src_hbm, dst_hbm, out_hbm, idx_s, idx_d, bufs, zbuf,
               acc, sem_g, sem_s):
    c = lax.axis_index("c")
    s = lax.axis_index("s")

    _zero_vmem(zbuf, Z, FH)
    for t in range((NZ + NSUB - 1) // NSUB):
        j = s + NSUB * t

        @pl.when(j < NZ)
        def _():
            pltpu.sync_copy(zbuf, acc.at[pl.ds(j * Z, Z)])
    plsc.subcore_barrier()

    for half in range(2):
        pltpu.sync_copy(src_hbm.at[c, s, pl.ds(half * NHALF, NHALF)], idx_s)
        pltpu.sync_copy(dst_hbm.at[s, pl.ds(half * NHALF, NHALF)], idx_d)
        for b in range(NBUF):
            pltpu.async_copy(h_hbm.at[idx_s.at[b]], bufs[b], sem_g)

        def chunk(i, carry):
            for b in range(NBUF):
                k = NBUF * i + b
                pltpu.make_async_copy(h_hbm.at[idx_s.at[k]], bufs[b],
                                      sem_g).wait()
                pltpu.async_copy(bufs[b], acc.at[idx_d.at[k]], sem_s,
                                 add=True)
            for b in range(NBUF):
                k = NBUF * i + b
                pltpu.make_async_copy(bufs[b], acc.at[idx_d.at[k]],
                                      sem_s).wait()

                @pl.when(k + NBUF < NHALF)
                def _():
                    pltpu.async_copy(h_hbm.at[idx_s.at[k + NBUF]], bufs[b],
                                     sem_g)
            return carry
        lax.fori_loop(0, NHALF // NBUF, chunk, None)

    plsc.subcore_barrier()
    for t in range((NZ + NSUB - 1) // NSUB):
        j = s + NSUB * t

        @pl.when(j < NZ)
        def _():
            pltpu.sync_copy(acc.at[pl.ds(j * Z, Z)],
                            out_hbm.at[c, pl.ds(j * Z, Z)])


_sc_spmm = pl.kernel(
    _spmm_body,
    out_type=jax.ShapeDtypeStruct((NCORES, N, FH), jnp.float32),
    mesh=_mesh,
    scratch_types=[
        pltpu.VMEM((NHALF, CS), jnp.int32),
        pltpu.VMEM((NHALF, CS), jnp.int32),
        tuple(pltpu.VMEM((CS, FH), jnp.float32) for _ in range(NBUF)),
        pltpu.VMEM((Z, FH), jnp.float32),
        pltpu.VMEM_SHARED((N + 8, FH), jnp.float32),
        pltpu.SemaphoreType.DMA,
        pltpu.SemaphoreType.DMA,
    ],
    compiler_params=pltpu.CompilerParams(use_tc_tiling_on_sc=False))

NCHUNK_D = 80
WIN = 8


def _cnt_body(dst_hbm, cnt_hbm, idx_d, ones, zbuf16, cnt_sh, sem):
    c = lax.axis_index("c")
    s = lax.axis_index("s")
    wid = c * NSUB + s
    pltpu.sync_copy(dst_hbm.at[wid], idx_d)

    _zero_vmem(zbuf16, Z, CW)

    def fill_ones(i, carry):
        ones[i, pl.ds(0, CW)] = jnp.ones((CW,), jnp.float32)
        return carry
    lax.fori_loop(0, C, fill_ones, None)
    for t in range((NZ + NSUB - 1) // NSUB):
        j = s + NSUB * t

        @pl.when(j < NZ)
        def _():
            pltpu.sync_copy(zbuf16, cnt_sh.at[pl.ds(j * Z, Z)])
    plsc.subcore_barrier()

    def chunk(k, carry):
        pltpu.async_copy(ones, cnt_sh.at[idx_d.at[k]], sem, add=True)

        @pl.when(k >= WIN)
        def _():
            pltpu.make_async_copy(ones, cnt_sh.at[idx_d.at[0]], sem).wait()
        return carry
    lax.fori_loop(0, NCHUNK_D, chunk, None)
    for _ in range(WIN):
        pltpu.make_async_copy(ones, cnt_sh.at[idx_d.at[0]], sem).wait()

    plsc.subcore_barrier()
    for t in range((NZ + NSUB - 1) // NSUB):
        j = s + NSUB * t

        @pl.when(j < NZ)
        def _():
            pltpu.sync_copy(cnt_sh.at[pl.ds(j * Z, Z)],
                            cnt_hbm.at[c, pl.ds(j * Z, Z)])


_sc_cnt = pl.kernel(
    _cnt_body,
    out_type=jax.ShapeDtypeStruct((NCORES, N, CW), jnp.float32),
    mesh=_mesh,
    scratch_types=[
        pltpu.VMEM((NCHUNK_D, C), jnp.int32),
        pltpu.VMEM((C, CW), jnp.float32),
        pltpu.VMEM((Z, CW), jnp.float32),
        pltpu.VMEM_SHARED((N + 8, CW), jnp.float32),
        pltpu.SemaphoreType.DMA,
    ],
    compiler_params=pltpu.CompilerParams(use_tc_tiling_on_sc=False))

R = 2000


def _dots(mean, h, wl_ref, wr_ref, bl_ref):
    return (jnp.dot(mean, wl_ref[...], preferred_element_type=jnp.float32,
                    precision=lax.Precision.DEFAULT)
            + jnp.dot(h, wr_ref[...], preferred_element_type=jnp.float32,
                      precision=lax.Precision.DEFAULT)
            + bl_ref[...])


def _combine_body(p_ref, c_ref, h_ref, wl_ref, wr_ref, bl_ref, o_ref):
    cnt = jnp.maximum(c_ref[0, :, 0:1] + c_ref[1, :, 0:1], 1.0)
    mean = jnp.concatenate([p_ref[0], p_ref[1]], axis=1) / cnt
    o_ref[...] = jnp.maximum(_dots(mean, h_ref[...], wl_ref, wr_ref, bl_ref),
                             0.0)


def _final_body(p_ref, c_ref, h_ref, wl_ref, wr_ref, bl_ref, wc_ref, bc_ref,
                o_ref):
    cnt = jnp.maximum(c_ref[0, :, 0:1] + c_ref[1, :, 0:1], 1.0)
    mean = jnp.concatenate([p_ref[0], p_ref[1]], axis=1) / cnt
    y = _dots(mean, h_ref[...], wl_ref, wr_ref, bl_ref)
    o_ref[...] = (jnp.dot(y, wc_ref[...], preferred_element_type=jnp.float32,
                          precision=lax.Precision.DEFAULT)
                  + bc_ref[...])


_common_specs = [
    pl.BlockSpec((NCORES, R, FH), lambda i: (0, i, 0)),
    pl.BlockSpec((NCORES, R, CW), lambda i: (0, i, 0)),
    pl.BlockSpec((R, F), lambda i: (i, 0)),
    pl.BlockSpec((F, F), lambda i: (0, 0)),
    pl.BlockSpec((F, F), lambda i: (0, 0)),
    pl.BlockSpec((1, F), lambda i: (0, 0)),
]

_combine_relu = pl.pallas_call(
    _combine_body,
    grid=(N // R,),
    in_specs=_common_specs,
    out_specs=pl.BlockSpec((R, F), lambda i: (i, 0)),
    out_shape=jax.ShapeDtypeStruct((N, F), jnp.float32),
)

_combine_final = pl.pallas_call(
    _final_body,
    grid=(N // R,),
    in_specs=_common_specs + [
        pl.BlockSpec((F, NCLS), lambda i: (0, 0)),
        pl.BlockSpec((1, NCLS), lambda i: (0, 0)),
    ],
    out_specs=pl.BlockSpec((R, NCLS), lambda i: (i, 0)),
    out_shape=jax.ShapeDtypeStruct((N, NCLS), jnp.float32),
)


def kernel(x, edge_index, Wl0, bl0, Wr0, Wl1, bl1, Wr1, Wl2, bl2, Wr2, Wc,
           bc):
    srcw = edge_index[0].reshape(NSUB, EPS)
    src2 = jnp.stack([2 * srcw, 2 * srcw + 1]).reshape(NCORES, NSUB,
                                                       NCHUNK, CS)
    dst3 = edge_index[1].reshape(NSUB, NCHUNK, CS)
    dw = edge_index[1].reshape(NCORES * NSUB, E // (NCORES * NSUB))
    dwp = jnp.pad(dw, ((0, 0), (0, NCHUNK_D * C - dw.shape[1])),
                  constant_values=N).reshape(NCORES * NSUB, NCHUNK_D, C)

    cnt = _sc_cnt(dwp)
    p0 = _sc_spmm(x.reshape(2 * N, FH), src2, dst3)
    h1 = _combine_relu(p0, cnt, x, Wl0, Wr0, bl0.reshape(1, F))
    p1 = _sc_spmm(h1.reshape(2 * N, FH), src2, dst3)
    h2 = _combine_relu(p1, cnt, h1, Wl1, Wr1, bl1.reshape(1, F))
    p2 = _sc_spmm(h2.reshape(2 * N, FH), src2, dst3)
    return _combine_final(p2, cnt, h2, Wl2, Wr2, bl2.reshape(1, F), Wc,
                          bc.reshape(1, NCLS))


# speedup vs baseline: 3.0269x; 1.0349x over previous
import functools

import jax
import jax.numpy as jnp
from jax import lax
from jax.experimental import pallas as pl
from jax.experimental.pallas import tpu as pltpu
from jax.experimental.pallas import tpu_sc as plsc

N = 10000
E = 320000
F = 128
FH = F // 2
NCLS = 64

NCORES = 2
NSUB = 16
EPS = E // NSUB
C = 128
CS = 125
NCHUNK = EPS // CS
NHALF = NCHUNK // 2
Z = 200
NZ = N // Z
ZC = 104
CW = 16

_mesh = plsc.VectorSubcoreMesh(core_axis_name="c", subcore_axis_name="s")


def _zero_vmem(ref, nrow, ncol):
    def body(i, carry):
        ref[i // (ncol // 16), pl.ds((i % (ncol // 16)) * 16, 16)] = (
            jnp.zeros((16,), jnp.float32))
        return carry
    lax.fori_loop(0, nrow * (ncol // 16), body, None)


NBUF = 5


def _spmm_body(h_hbm, src_hbm, dst_hbm, out_hbm, idx_s, idx_d, bufs, zbuf,
               acc, sem_g, sem_s):
    c = lax.axis_index("c")
    s = lax.axis_index("s")

    pltpu.sync_copy(src_hbm.at[c, s, pl.ds(0, NHALF)], idx_s)
    pltpu.sync_copy(dst_hbm.at[s, pl.ds(0, NHALF)], idx_d)
    for b in range(NBUF):
        pltpu.async_copy(h_hbm.at[idx_s.at[b]], bufs[b], sem_g)

    _zero_vmem(zbuf, ZC, FH)
    for t in range(6):
        pltpu.sync_copy(zbuf, acc.at[pl.ds((s + NSUB * t) * ZC, ZC)])

    @pl.when(s == 0)
    def _():
        pltpu.sync_copy(zbuf.at[pl.ds(0, 16)], acc.at[pl.ds(96 * ZC, 16)])
    plsc.subcore_barrier()

    for half in range(2):
        if half:
            pltpu.sync_copy(src_hbm.at[c, s, pl.ds(NHALF, NHALF)], idx_s)
            pltpu.sync_copy(dst_hbm.at[s, pl.ds(NHALF, NHALF)], idx_d)
            for b in range(NBUF):
                pltpu.async_copy(h_hbm.at[idx_s.at[b]], bufs[b], sem_g)

        def chunk(i, carry):
            for b in range(NBUF):
                k = NBUF * i + b
                pltpu.make_async_copy(h_hbm.at[idx_s.at[k]], bufs[b],
                                      sem_g).wait()
                pltpu.async_copy(bufs[b], acc.at[idx_d.at[k]], sem_s,
                                 add=True)
            for b in range(NBUF):
                k = NBUF * i + b
                pltpu.make_async_copy(bufs[b], acc.at[idx_d.at[k]],
                                      sem_s).wait()

                @pl.when(k + NBUF < NHALF)
                def _():
                    pltpu.async_copy(h_hbm.at[idx_s.at[k + NBUF]], bufs[b],
                                     sem_g)
            return carry
        lax.fori_loop(0, NHALF // NBUF, chunk, None)

    plsc.subcore_barrier()
    for t in range((NZ + NSUB - 1) // NSUB):
        j = s + NSUB * t

        @pl.when(j < NZ)
        def _():
            pltpu.sync_copy(acc.at[pl.ds(j * Z, Z)],
                            out_hbm.at[c, pl.ds(j * Z, Z)])


_sc_spmm = pl.kernel(
    _spmm_body,
    out_type=jax.ShapeDtypeStruct((NCORES, N, FH), jnp.float32),
    mesh=_mesh,
    scratch_types=[
        pltpu.VMEM((NHALF, CS), jnp.int32),
        pltpu.VMEM((NHALF, CS), jnp.int32),
        tuple(pltpu.VMEM((CS, FH), jnp.float32) for _ in range(NBUF)),
        pltpu.VMEM((ZC, FH), jnp.float32),
        pltpu.VMEM_SHARED((N + 8, FH), jnp.float32),
        pltpu.SemaphoreType.DMA,
        pltpu.SemaphoreType.DMA,
    ],
    compiler_params=pltpu.CompilerParams(use_tc_tiling_on_sc=False))

NCHUNK_D = 80
WIN = 8


def _cnt_body(dst_hbm, cnt_hbm, idx_d, ones, zbuf16, cnt_sh, sem):
    c = lax.axis_index("c")
    s = lax.axis_index("s")
    wid = c * NSUB + s
    pltpu.sync_copy(dst_hbm.at[wid], idx_d)

    _zero_vmem(zbuf16, Z, CW)

    def fill_ones(i, carry):
        ones[i, pl.ds(0, CW)] = jnp.ones((CW,), jnp.float32)
        return carry
    lax.fori_loop(0, C, fill_ones, None)
    for t in range((NZ + NSUB - 1) // NSUB):
        j = s + NSUB * t

        @pl.when(j < NZ)
        def _():
            pltpu.sync_copy(zbuf16, cnt_sh.at[pl.ds(j * Z, Z)])
    plsc.subcore_barrier()

    def chunk(k, carry):
        pltpu.async_copy(ones, cnt_sh.at[idx_d.at[k]], sem, add=True)

        @pl.when(k >= WIN)
        def _():
            pltpu.make_async_copy(ones, cnt_sh.at[idx_d.at[0]], sem).wait()
        return carry
    lax.fori_loop(0, NCHUNK_D, chunk, None)
    for _ in range(WIN):
        pltpu.make_async_copy(ones, cnt_sh.at[idx_d.at[0]], sem).wait()

    plsc.subcore_barrier()
    for t in range((NZ + NSUB - 1) // NSUB):
        j = s + NSUB * t

        @pl.when(j < NZ)
        def _():
            pltpu.sync_copy(cnt_sh.at[pl.ds(j * Z, Z)],
                            cnt_hbm.at[c, pl.ds(j * Z, Z)])


_sc_cnt = pl.kernel(
    _cnt_body,
    out_type=jax.ShapeDtypeStruct((NCORES, N, CW), jnp.float32),
    mesh=_mesh,
    scratch_types=[
        pltpu.VMEM((NCHUNK_D, C), jnp.int32),
        pltpu.VMEM((C, CW), jnp.float32),
        pltpu.VMEM((Z, CW), jnp.float32),
        pltpu.VMEM_SHARED((N + 8, CW), jnp.float32),
        pltpu.SemaphoreType.DMA,
    ],
    compiler_params=pltpu.CompilerParams(use_tc_tiling_on_sc=False))

R = 2000


def _dots(mean, h, wl_ref, wr_ref, bl_ref):
    return (jnp.dot(mean, wl_ref[...], preferred_element_type=jnp.float32,
                    precision=lax.Precision.DEFAULT)
            + jnp.dot(h, wr_ref[...], preferred_element_type=jnp.float32,
                      precision=lax.Precision.DEFAULT)
            + bl_ref[...])


def _combine_body(p_ref, c_ref, h_ref, wl_ref, wr_ref, bl_ref, o_ref):
    cnt = jnp.maximum(c_ref[0, :, 0:1] + c_ref[1, :, 0:1], 1.0)
    mean = jnp.concatenate([p_ref[0], p_ref[1]], axis=1) / cnt
    o_ref[...] = jnp.maximum(_dots(mean, h_ref[...], wl_ref, wr_ref, bl_ref),
                             0.0)


def _final_body(p_ref, c_ref, h_ref, wl_ref, wr_ref, bl_ref, wc_ref, bc_ref,
                o_ref):
    cnt = jnp.maximum(c_ref[0, :, 0:1] + c_ref[1, :, 0:1], 1.0)
    mean = jnp.concatenate([p_ref[0], p_ref[1]], axis=1) / cnt
    y = _dots(mean, h_ref[...], wl_ref, wr_ref, bl_ref)
    o_ref[...] = (jnp.dot(y, wc_ref[...], preferred_element_type=jnp.float32,
                          precision=lax.Precision.DEFAULT)
                  + bc_ref[...])


_common_specs = [
    pl.BlockSpec((NCORES, R, FH), lambda i: (0, i, 0)),
    pl.BlockSpec((NCORES, R, CW), lambda i: (0, i, 0)),
    pl.BlockSpec((R, F), lambda i: (i, 0)),
    pl.BlockSpec((F, F), lambda i: (0, 0)),
    pl.BlockSpec((F, F), lambda i: (0, 0)),
    pl.BlockSpec((1, F), lambda i: (0, 0)),
]

_combine_relu = pl.pallas_call(
    _combine_body,
    grid=(N // R,),
    in_specs=_common_specs,
    out_specs=pl.BlockSpec((R, F), lambda i: (i, 0)),
    out_shape=jax.ShapeDtypeStruct((N, F), jnp.float32),
)

_combine_final = pl.pallas_call(
    _final_body,
    grid=(N // R,),
    in_specs=_common_specs + [
        pl.BlockSpec((F, NCLS), lambda i: (0, 0)),
        pl.BlockSpec((1, NCLS), lambda i: (0, 0)),
    ],
    out_specs=pl.BlockSpec((R, NCLS), lambda i: (i, 0)),
    out_shape=jax.ShapeDtypeStruct((N, NCLS), jnp.float32),
)


def kernel(x, edge_index, Wl0, bl0, Wr0, Wl1, bl1, Wr1, Wl2, bl2, Wr2, Wc,
           bc):
    srcw = edge_index[0].reshape(NSUB, EPS)
    src2 = jnp.stack([2 * srcw, 2 * srcw + 1]).reshape(NCORES, NSUB,
                                                       NCHUNK, CS)
    dst3 = edge_index[1].reshape(NSUB, NCHUNK, CS)
    dw = edge_index[1].reshape(NCORES * NSUB, E // (NCORES * NSUB))
    dwp = jnp.pad(dw, ((0, 0), (0, NCHUNK_D * C - dw.shape[1])),
                  constant_values=N).reshape(NCORES * NSUB, NCHUNK_D, C)

    cnt = _sc_cnt(dwp)
    p0 = _sc_spmm(x.reshape(2 * N, FH), src2, dst3)
    h1 = _combine_relu(p0, cnt, x, Wl0, Wr0, bl0.reshape(1, F))
    p1 = _sc_spmm(h1.reshape(2 * N, FH), src2, dst3)
    h2 = _combine_relu(p1, cnt, h1, Wl1, Wr1, bl1.reshape(1, F))
    p2 = _sc_spmm(h2.reshape(2 * N, FH), src2, dst3)
    return _combine_final(p2, cnt, h2, Wl2, Wr2, bl2.reshape(1, F), Wc,
                          bc.reshape(1, NCLS))
